# baseline (speedup 1.0000x reference)
import functools

import jax
import jax.numpy as jnp
from jax import lax
from jax.experimental import pallas as pl
from jax.experimental.pallas import tpu as pltpu
from jax.experimental.pallas import tpu_sc as plsc

N_NODES = 10000
N_EDGES = 320000
D_IN = 128
D_HID = 16

NC = 2
NS = 16
NW = NC * NS
EPW = N_EDGES // NW
CH = 125
NCH = EPW // CH
N_PAD = 10240
RPT = N_PAD // NS
MCH = 16
NMG = NCH // MCH
MGE = MCH * CH

_mesh = plsc.VectorSubcoreMesh(core_axis_name="c", subcore_axis_name="s")
_sc_params = pltpu.CompilerParams(use_tc_tiling_on_sc=False)


@functools.partial(
    pl.kernel,
    out_type=jax.ShapeDtypeStruct((NC, N_PAD), jnp.float32),
    mesh=_mesh,
    scratch_types=[
        pltpu.VMEM((NMG, MGE), jnp.int32),
        pltpu.VMEM((MGE,), jnp.float32),
        pltpu.VMEM_SHARED((N_PAD,), jnp.float32),
        pltpu.SemaphoreType.DMA,
    ],
    compiler_params=_sc_params,
)
def _deg_pass(dst_hbm, ones_hbm, zeros_hbm, out_hbm, dstv, onesv, acc, sem):
    cid = lax.axis_index("c")
    sid = lax.axis_index("s")
    wid = cid * NS + sid
    pltpu.sync_copy(zeros_hbm.at[pl.ds(sid * RPT, RPT)],
                    acc.at[pl.ds(sid * RPT, RPT)])
    pltpu.sync_copy(ones_hbm, onesv)
    pltpu.sync_copy(dst_hbm.at[wid], dstv)
    plsc.subcore_barrier()

    for m in range(NMG):
        pltpu.async_copy(onesv, acc.at[dstv.at[m]], sem, add=True)
    for m in range(NMG):
        pltpu.make_async_copy(onesv, acc.at[dstv.at[m]], sem).wait()
    plsc.subcore_barrier()
    pltpu.sync_copy(acc.at[pl.ds(sid * RPT, RPT)],
                    out_hbm.at[cid, pl.ds(sid * RPT, RPT)])


@functools.partial(
    pl.kernel,
    out_type=jax.ShapeDtypeStruct((NC, N_PAD, D_HID), jnp.float32),
    mesh=_mesh,
    scratch_types=[
        pltpu.VMEM((NMG, MGE), jnp.int32),
        pltpu.VMEM((NMG, MGE), jnp.int32),
        pltpu.VMEM((MGE, D_HID), jnp.float32),
        pltpu.VMEM((MGE, D_HID), jnp.float32),
        pltpu.VMEM((MGE, D_HID), jnp.float32),
        pltpu.VMEM_SHARED((N_PAD, D_HID), jnp.float32),
        [pltpu.SemaphoreType.DMA] * 3,
        [pltpu.SemaphoreType.DMA] * 3,
    ],
    compiler_params=_sc_params,
)
def _agg_pass(table_hbm, src_hbm, dst_hbm, zeros_hbm, out_hbm,
              srcv, dstv, buf0, buf1, buf2, acc, sg, ss):
    cid = lax.axis_index("c")
    sid = lax.axis_index("s")
    wid = cid * NS + sid
    pltpu.sync_copy(zeros_hbm.at[pl.ds(sid * RPT, RPT)],
                    acc.at[pl.ds(sid * RPT, RPT)])
    pltpu.sync_copy(src_hbm.at[wid], srcv)
    pltpu.sync_copy(dst_hbm.at[wid], dstv)
    plsc.subcore_barrier()

    bufs = [buf0, buf1, buf2]

    def _fire_gather(m):
        s = m % 3
        pltpu.async_copy(table_hbm.at[srcv.at[m]], bufs[s], sg[s])

    def _drain_gather(m):
        s = m % 3
        pltpu.make_async_copy(table_hbm.at[srcv.at[m]], bufs[s],
                              sg[s]).wait()

    def _fire_scatter(m):
        s = m % 3
        pltpu.async_copy(bufs[s], acc.at[dstv.at[m]], ss[s], add=True)

    def _drain_scatter(m):
        s = m % 3
        pltpu.make_async_copy(bufs[s], acc.at[dstv.at[m]], ss[s]).wait()

    _fire_gather(0)
    _fire_gather(1)
    for m in range(NMG):
        _drain_gather(m)
        _fire_scatter(m)
        if m + 2 < NMG:
            if m >= 1:
                _drain_scatter(m - 1)
            _fire_gather(m + 2)
    for m in range(NMG - 3, NMG):
        _drain_scatter(m)

    plsc.subcore_barrier()
    pltpu.sync_copy(acc.at[pl.ds(sid * RPT, RPT)],
                    out_hbm.at[cid, pl.ds(sid * RPT, RPT)])


def _scale_body(x_ref, w_ref, dp_ref, o_ref, dinv_ref):
    deg = dp_ref[0] + dp_ref[1] + 1.0
    dinv = lax.rsqrt(deg).reshape(N_PAD, 1)
    dinv_ref[...] = dinv
    h1 = jnp.dot(x_ref[...], w_ref[...], preferred_element_type=jnp.float32)
    o_ref[0:N_NODES, :] = h1 * dinv[0:N_NODES]
    o_ref[N_NODES:N_PAD, :] = jnp.zeros((N_PAD - N_NODES, D_HID), jnp.float32)


def _layer1_body(pp_ref, hn_ref, dinv_ref, b1_ref, o_ref):
    dinv = dinv_ref[...]
    hn = hn_ref[...]
    agg = pp_ref[0] + pp_ref[1] + hn
    r = jnp.maximum(agg * dinv + b1_ref[...], 0.0)
    o_ref[...] = r * dinv


def _layer2_body(pp_ref, rn_ref, dinv_ref, w2_ref, b2_ref, o_ref):
    agg = (pp_ref[0][0:N_NODES, :] + pp_ref[1][0:N_NODES, :]
           + rn_ref[0:N_NODES, :])
    out = jnp.dot(agg, w2_ref[...], preferred_element_type=jnp.float32)
    o_ref[...] = out * dinv_ref[0:N_NODES, :] + b2_ref[...]


@jax.jit
def kernel(x, edge_index, W1, b1, W2, b2):
    if edge_index.dtype != jnp.int32:
        edge_index = edge_index.astype(jnp.int32)
    src = edge_index[0].reshape(NW, NMG, MGE)
    dst = edge_index[1].reshape(NW, NMG, MGE)
    ones_h = jnp.ones((MGE,), jnp.float32)
    zeros1 = jnp.zeros((N_PAD,), jnp.float32)
    zeros2 = jnp.zeros((N_PAD, D_HID), jnp.float32)

    degp = _deg_pass(dst, ones_h, zeros1)

    hn1, dinv = pl.pallas_call(
        _scale_body,
        out_shape=[jax.ShapeDtypeStruct((N_PAD, D_HID), jnp.float32),
                   jax.ShapeDtypeStruct((N_PAD, 1), jnp.float32)],
    )(x, W1, degp)

    pp1 = _agg_pass(hn1, src, dst, zeros2)

    rn = pl.pallas_call(
        _layer1_body,
        out_shape=jax.ShapeDtypeStruct((N_PAD, D_HID), jnp.float32),
    )(pp1, hn1, dinv, b1.reshape(1, D_HID))

    pp2 = _agg_pass(rn, src, dst, zeros2)

    return pl.pallas_call(
        _layer2_body,
        out_shape=jax.ShapeDtypeStruct((N_NODES, 1), jnp.float32),
    )(pp2, rn, dinv, W2, b2.reshape(1, 1))


# speedup vs baseline: 1.0845x; 1.0845x over previous
import functools

import jax
import jax.numpy as jnp
from jax import lax
from jax.experimental import pallas as pl
from jax.experimental.pallas import tpu as pltpu
from jax.experimental.pallas import tpu_sc as plsc

N_NODES = 10000
N_EDGES = 320000
D_IN = 128
D_HID = 16

NC = 2
NS = 16
NW = NC * NS
EPW = N_EDGES // NW
CH = 125
NCH = EPW // CH
N_PAD = 10240
RPT = N_PAD // NS
MCH = 16
NMG = NCH // MCH
MGE = MCH * CH

_mesh = plsc.VectorSubcoreMesh(core_axis_name="c", subcore_axis_name="s")
_sc_params = pltpu.CompilerParams(use_tc_tiling_on_sc=False)


@functools.partial(
    pl.kernel,
    out_type=jax.ShapeDtypeStruct((NC, N_PAD), jnp.float32),
    mesh=_mesh,
    scratch_types=[
        pltpu.VMEM((NMG, MGE), jnp.int32),
        pltpu.VMEM((MGE,), jnp.float32),
        pltpu.VMEM_SHARED((N_PAD,), jnp.float32),
        pltpu.SemaphoreType.DMA,
    ],
    compiler_params=_sc_params,
)
def _deg_pass(edges_hbm, ones_hbm, zeros_hbm, out_hbm, dstv, onesv, acc, sem):
    cid = lax.axis_index("c")
    sid = lax.axis_index("s")
    wid = cid * NS + sid
    pltpu.sync_copy(zeros_hbm.at[pl.ds(sid * RPT, RPT)],
                    acc.at[pl.ds(sid * RPT, RPT)])
    pltpu.sync_copy(ones_hbm, onesv)
    pltpu.sync_copy(edges_hbm.at[1, wid], dstv)
    plsc.subcore_barrier()

    for m in range(NMG):
        pltpu.async_copy(onesv, acc.at[dstv.at[m]], sem, add=True)
    for m in range(NMG):
        pltpu.make_async_copy(onesv, acc.at[dstv.at[m]], sem).wait()
    plsc.subcore_barrier()
    pltpu.sync_copy(acc.at[pl.ds(sid * RPT, RPT)],
                    out_hbm.at[cid, pl.ds(sid * RPT, RPT)])


@functools.partial(
    pl.kernel,
    out_type=jax.ShapeDtypeStruct((NC, N_PAD, D_HID), jnp.float32),
    mesh=_mesh,
    scratch_types=[
        pltpu.VMEM((NMG, MGE), jnp.int32),
        pltpu.VMEM((NMG, MGE), jnp.int32),
        pltpu.VMEM((MGE, D_HID), jnp.float32),
        pltpu.VMEM((MGE, D_HID), jnp.float32),
        pltpu.VMEM((MGE, D_HID), jnp.float32),
        pltpu.VMEM_SHARED((N_PAD, D_HID), jnp.float32),
        [pltpu.SemaphoreType.DMA] * 3,
        [pltpu.SemaphoreType.DMA] * 3,
    ],
    compiler_params=_sc_params,
)
def _agg_pass(table_hbm, edges_hbm, zeros_hbm, out_hbm,
              srcv, dstv, buf0, buf1, buf2, acc, sg, ss):
    cid = lax.axis_index("c")
    sid = lax.axis_index("s")
    wid = cid * NS + sid
    pltpu.sync_copy(zeros_hbm.at[pl.ds(sid * RPT, RPT)],
                    acc.at[pl.ds(sid * RPT, RPT)])
    pltpu.sync_copy(edges_hbm.at[0, wid], srcv)
    pltpu.sync_copy(edges_hbm.at[1, wid], dstv)
    plsc.subcore_barrier()

    bufs = [buf0, buf1, buf2]

    def _fire_gather(m):
        s = m % 3
        pltpu.async_copy(table_hbm.at[srcv.at[m]], bufs[s], sg[s])

    def _drain_gather(m):
        s = m % 3
        pltpu.make_async_copy(table_hbm.at[srcv.at[m]], bufs[s],
                              sg[s]).wait()

    def _fire_scatter(m):
        s = m % 3
        pltpu.async_copy(bufs[s], acc.at[dstv.at[m]], ss[s], add=True)

    def _drain_scatter(m):
        s = m % 3
        pltpu.make_async_copy(bufs[s], acc.at[dstv.at[m]], ss[s]).wait()

    _fire_gather(0)
    _fire_gather(1)
    for m in range(NMG):
        _drain_gather(m)
        _fire_scatter(m)
        if m + 2 < NMG:
            if m >= 1:
                _drain_scatter(m - 1)
            _fire_gather(m + 2)
    for m in range(NMG - 3, NMG):
        _drain_scatter(m)

    plsc.subcore_barrier()
    pltpu.sync_copy(acc.at[pl.ds(sid * RPT, RPT)],
                    out_hbm.at[cid, pl.ds(sid * RPT, RPT)])


def _scale_body(x_ref, w_ref, dp_ref, o_ref, dinv_ref):
    deg = dp_ref[0] + dp_ref[1] + 1.0
    dinv = lax.rsqrt(deg).reshape(N_PAD, 1)
    dinv_ref[...] = dinv
    h1 = jnp.dot(x_ref[...], w_ref[...], preferred_element_type=jnp.float32)
    o_ref[0:N_NODES, :] = h1 * dinv[0:N_NODES]
    o_ref[N_NODES:N_PAD, :] = jnp.zeros((N_PAD - N_NODES, D_HID), jnp.float32)


def _layer1_body(pp_ref, hn_ref, dinv_ref, b1_ref, o_ref):
    dinv = dinv_ref[...]
    hn = hn_ref[...]
    agg = pp_ref[0] + pp_ref[1] + hn
    r = jnp.maximum(agg * dinv + b1_ref[...], 0.0)
    o_ref[...] = r * dinv


def _layer2_body(pp_ref, rn_ref, dinv_ref, w2_ref, b2_ref, o_ref):
    agg = (pp_ref[0][0:N_NODES, :] + pp_ref[1][0:N_NODES, :]
           + rn_ref[0:N_NODES, :])
    out = jnp.dot(agg, w2_ref[...], preferred_element_type=jnp.float32)
    o_ref[...] = out * dinv_ref[0:N_NODES, :] + b2_ref[...]


@jax.jit
def kernel(x, edge_index, W1, b1, W2, b2):
    if edge_index.dtype != jnp.int32:
        edge_index = edge_index.astype(jnp.int32)
    edges = edge_index.reshape(2, NW, NMG, MGE)
    ones_h = jnp.ones((MGE,), jnp.float32)
    zeros1 = jnp.zeros((N_PAD,), jnp.float32)
    zeros2 = jnp.zeros((N_PAD, D_HID), jnp.float32)

    degp = _deg_pass(edges, ones_h, zeros1)

    hn1, dinv = pl.pallas_call(
        _scale_body,
        out_shape=[jax.ShapeDtypeStruct((N_PAD, D_HID), jnp.float32),
                   jax.ShapeDtypeStruct((N_PAD, 1), jnp.float32)],
    )(x, W1, degp)

    pp1 = _agg_pass(hn1, edges, zeros2)

    rn = pl.pallas_call(
        _layer1_body,
        out_shape=jax.ShapeDtypeStruct((N_PAD, D_HID), jnp.float32),
    )(pp1, hn1, dinv, b1.reshape(1, D_HID))

    pp2 = _agg_pass(rn, edges, zeros2)

    return pl.pallas_call(
        _layer2_body,
        out_shape=jax.ShapeDtypeStruct((N_NODES, 1), jnp.float32),
    )(pp2, rn, dinv, W2, b2.reshape(1, 1))
